# baseline (speedup 1.0000x reference)
import jax
import jax.numpy as jnp
from jax import lax
from jax.experimental import pallas as pl
from jax.experimental.pallas import tpu as pltpu
from jax.experimental.pallas import tpu_sc as plsc

N = 10000
E = 320000
D = 128
H = 128
OUT = 128
G = 64
HH = H // 2

BN = 1000
NB = N // BN

NC = 2
NS = 16
EPT = E // NS
CHUNK = 500
NCHUNK = EPT // CHUNK
KB = 4
NBLK = NCHUNK // KB
ZROWS = 125
ZPT = N // NS


def _mm1(h_ref, w_ref, o_ref):
    o_ref[...] = jnp.dot(h_ref[...], w_ref[...],
                         preferred_element_type=jnp.float32)


def _hm_mid(sp_ref, n_ref, w2_ref, h_out_ref, m_ref):
    n = jnp.concatenate([n_ref[0], n_ref[1]], axis=1)
    h = jnp.maximum(sp_ref[...] + n, 0.0)
    h_out_ref[...] = h
    m_ref[...] = jnp.dot(h, w2_ref[...], preferred_element_type=jnp.float32)


def _final(sp_ref, n_ref, b_ref, c1w_ref, c1b_ref, c2w_ref, c2b_ref,
           out_ref, pooled):
    i = pl.program_id(0)
    n = jnp.concatenate([n_ref[0], n_ref[1]], axis=1)
    h = jnp.maximum(sp_ref[...] + n, 0.0)
    b = b_ref[0, 0, :]
    onehot = (lax.broadcasted_iota(jnp.int32, (G, BN), 0) ==
              b[None, :]).astype(jnp.float32)
    part = jnp.dot(onehot, h, preferred_element_type=jnp.float32)

    @pl.when(i == 0)
    def _():
        pooled[...] = part

    @pl.when(i > 0)
    def _():
        pooled[...] += part

    @pl.when(i == NB - 1)
    def _():
        g = jnp.maximum(
            jnp.dot(pooled[...], c1w_ref[...],
                    preferred_element_type=jnp.float32) + c1b_ref[...], 0.0)
        out_ref[...] = jnp.dot(
            g, c2w_ref[...], preferred_element_type=jnp.float32) + c2b_ref[...]


def _tc_mm1(h, w):
    return pl.pallas_call(
        _mm1,
        grid=(NB,),
        in_specs=[
            pl.BlockSpec((BN, H), lambda i: (i, 0)),
            pl.BlockSpec((H, H), lambda i: (0, 0)),
        ],
        out_specs=pl.BlockSpec((BN, H), lambda i: (i, 0)),
        out_shape=jax.ShapeDtypeStruct((N, H), jnp.float32),
    )(h, w)


def _tc_hm_mid(s_prev, n2, w2):
    return pl.pallas_call(
        _hm_mid,
        grid=(NB,),
        in_specs=[
            pl.BlockSpec((BN, H), lambda i: (i, 0)),
            pl.BlockSpec((2, BN, HH), lambda i: (0, i, 0)),
            pl.BlockSpec((H, H), lambda i: (0, 0)),
        ],
        out_specs=[
            pl.BlockSpec((BN, H), lambda i: (i, 0)),
            pl.BlockSpec((BN, H), lambda i: (i, 0)),
        ],
        out_shape=[
            jax.ShapeDtypeStruct((N, H), jnp.float32),
            jax.ShapeDtypeStruct((N, H), jnp.float32),
        ],
    )(s_prev, n2, w2)


def _tc_final(s_prev, n2, batch3, c1w, c1b, c2w, c2b):
    return pl.pallas_call(
        _final,
        grid=(NB,),
        in_specs=[
            pl.BlockSpec((BN, H), lambda i: (i, 0)),
            pl.BlockSpec((2, BN, HH), lambda i: (0, i, 0)),
            pl.BlockSpec((1, 1, BN), lambda i: (i, 0, 0)),
            pl.BlockSpec((H, H), lambda i: (0, 0)),
            pl.BlockSpec((1, H), lambda i: (0, 0)),
            pl.BlockSpec((H, OUT), lambda i: (0, 0)),
            pl.BlockSpec((1, OUT), lambda i: (0, 0)),
        ],
        out_specs=pl.BlockSpec((G, OUT), lambda i: (0, 0)),
        out_shape=jax.ShapeDtypeStruct((G, OUT), jnp.float32),
        scratch_shapes=[pltpu.VMEM((G, H), jnp.float32)],
    )(s_prev, n2, batch3, c1w, c1b, c2w, c2b)


def _sc_agg_body(m2_hbm, pack_hbm, out_hbm,
                 idx_v, rows_v, zbuf, acc_sh,
                 sem_g0, sem_g1, sem_sc0, sem_sc1, sem_idx):
    c = lax.axis_index("c")
    s = lax.axis_index("s")

    zero16 = jnp.zeros((16,), jnp.float32)

    def _zfill(i, _):
        for k in range(HH // 16):
            zbuf[i, pl.ds(k * 16, 16)] = zero16
        return 0

    lax.fori_loop(0, ZROWS, _zfill, 0)

    tbl = m2_hbm
    sem_g = (sem_g0, sem_g1)
    sem_sc = (sem_sc0, sem_sc1)

    pltpu.sync_copy(pack_hbm.at[s, pl.ds(0, KB)], idx_v.at[0])
    pltpu.async_copy(tbl.at[idx_v.at[0, 0, c]], rows_v.at[0], sem_g0)

    def _zdma(j, _):
        pltpu.sync_copy(zbuf, acc_sh.at[pl.ds(s * ZPT + j * ZROWS, ZROWS)])
        return 0

    lax.fori_loop(0, ZPT // ZROWS, _zdma, 0)

    plsc.subcore_barrier()

    def _block(b, _):
        bp = b % 2
        bq = (b + 1) % 2
        for k in range(KB):
            p = k % 2
            q = 1 - p
            if k == 0:
                @pl.when(b > 0)
                def _():
                    pltpu.make_async_copy(
                        rows_v.at[q], acc_sh.at[idx_v.at[bp, k, 2]],
                        sem_sc[q]).wait()
            else:
                pltpu.make_async_copy(
                    rows_v.at[q], acc_sh.at[idx_v.at[bp, k, 2]],
                    sem_sc[q]).wait()
            if k == 1:
                @pl.when(b < NBLK - 1)
                def _():
                    pltpu.async_copy(
                        pack_hbm.at[s, pl.ds((b + 1) * KB, KB)],
                        idx_v.at[bq], sem_idx)
            if k < KB - 1:
                pltpu.async_copy(tbl.at[idx_v.at[bp, k + 1, c]],
                                 rows_v.at[q], sem_g[q])
            else:
                @pl.when(b < NBLK - 1)
                def _():
                    pltpu.make_async_copy(
                        pack_hbm.at[s, pl.ds((b + 1) * KB, KB)],
                        idx_v.at[bq], sem_idx).wait()
                    pltpu.async_copy(tbl.at[idx_v.at[bq, 0, c]],
                                     rows_v.at[q], sem_g[q])
            pltpu.make_async_copy(tbl.at[idx_v.at[bp, k, c]], rows_v.at[p],
                                  sem_g[p]).wait()
            pltpu.async_copy(rows_v.at[p], acc_sh.at[idx_v.at[bp, k, 2]],
                             sem_sc[p], add=True)
        return 0

    lax.fori_loop(0, NBLK, _block, 0)

    lastb = (NBLK - 1) % 2
    pltpu.make_async_copy(rows_v.at[1], acc_sh.at[idx_v.at[lastb, KB - 1, 2]],
                          sem_sc1).wait()

    plsc.subcore_barrier()

    pltpu.sync_copy(acc_sh.at[pl.ds(s * ZPT, ZPT)],
                    out_hbm.at[pl.ds(c * N + s * ZPT, ZPT)])


def _sc_aggregate(m2, pack):
    mesh = plsc.VectorSubcoreMesh(
        core_axis_name="c", subcore_axis_name="s",
        num_cores=NC, num_subcores=NS)
    f = pl.kernel(
        _sc_agg_body,
        out_type=jax.ShapeDtypeStruct((NC * N, HH), jnp.float32),
        mesh=mesh,
        compiler_params=pltpu.CompilerParams(use_tc_tiling_on_sc=False),
        scratch_types=[
            pltpu.VMEM((2, KB, 3, CHUNK), jnp.int32),
            pltpu.VMEM((2, CHUNK, HH), jnp.float32),
            pltpu.VMEM((ZROWS, HH), jnp.float32),
            pltpu.VMEM_SHARED((N, HH), jnp.float32),
            pltpu.SemaphoreType.DMA,
            pltpu.SemaphoreType.DMA,
            pltpu.SemaphoreType.DMA,
            pltpu.SemaphoreType.DMA,
            pltpu.SemaphoreType.DMA,
        ],
    )
    return f(m2, pack)


@jax.jit
def kernel(x, edge_index, batch, W1_0, W2_0, W1_1, W2_1, W1_2, W2_2,
           C1_w, C1_b, C2_w, C2_b):
    src_c = edge_index[0].reshape(NS, NCHUNK, CHUNK)
    pack = jnp.stack([2 * src_c, 2 * src_c + 1,
                      edge_index[1].reshape(NS, NCHUNK, CHUNK)],
                     axis=2)
    batch3 = batch.reshape(NB, 1, BN)
    c1b = C1_b.reshape(1, H)
    c2b = C2_b.reshape(1, OUT)

    m0 = _tc_mm1(x, W2_0)
    n0 = _sc_aggregate(m0.reshape(2 * N, HH), pack).reshape(2, N, HH)
    s0 = _tc_mm1(x, W1_0)
    h1, m1 = _tc_hm_mid(s0, n0, W2_1)
    n1 = _sc_aggregate(m1.reshape(2 * N, HH), pack).reshape(2, N, HH)
    s1 = _tc_mm1(h1, W1_1)
    h2, m2 = _tc_hm_mid(s1, n1, W2_2)
    n2 = _sc_aggregate(m2.reshape(2 * N, HH), pack).reshape(2, N, HH)
    s2 = _tc_mm1(h2, W1_2)
    return _tc_final(s2, n2, batch3, C1_w, c1b, C2_w, c2b)


# speedup vs baseline: 1.0113x; 1.0113x over previous
import jax
import jax.numpy as jnp
from jax import lax
from jax.experimental import pallas as pl
from jax.experimental.pallas import tpu as pltpu
from jax.experimental.pallas import tpu_sc as plsc

N = 10000
E = 320000
D = 128
H = 128
OUT = 128
G = 64
HH = H // 2

BN = 1000
NB = N // BN

NC = 2
NS = 16
EPT = E // NS
CHUNK = 500
NCHUNK = EPT // CHUNK
KB = 4
NBLK = NCHUNK // KB
ZROWS = 125
ZPT = N // NS


def _mm2_first(h_ref, w1_ref, w2_ref, s_ref, m_ref):
    h = h_ref[...]
    s_ref[...] = jnp.dot(h, w1_ref[...], preferred_element_type=jnp.float32)
    m_ref[...] = jnp.dot(h, w2_ref[...], preferred_element_type=jnp.float32)


def _mm2_mid(sp_ref, n_ref, w1_ref, w2_ref, s_ref, m_ref):
    n = jnp.concatenate([n_ref[0], n_ref[1]], axis=1)
    h = jnp.maximum(sp_ref[...] + n, 0.0)
    s_ref[...] = jnp.dot(h, w1_ref[...], preferred_element_type=jnp.float32)
    m_ref[...] = jnp.dot(h, w2_ref[...], preferred_element_type=jnp.float32)


def _final(sp_ref, n_ref, b_ref, c1w_ref, c1b_ref, c2w_ref, c2b_ref,
           out_ref, pooled):
    i = pl.program_id(0)
    n = jnp.concatenate([n_ref[0], n_ref[1]], axis=1)
    h = jnp.maximum(sp_ref[...] + n, 0.0)
    b = b_ref[0, 0, :]
    onehot = (lax.broadcasted_iota(jnp.int32, (G, BN), 0) ==
              b[None, :]).astype(jnp.float32)
    part = jnp.dot(onehot, h, preferred_element_type=jnp.float32)

    @pl.when(i == 0)
    def _():
        pooled[...] = part

    @pl.when(i > 0)
    def _():
        pooled[...] += part

    @pl.when(i == NB - 1)
    def _():
        g = jnp.maximum(
            jnp.dot(pooled[...], c1w_ref[...],
                    preferred_element_type=jnp.float32) + c1b_ref[...], 0.0)
        out_ref[...] = jnp.dot(
            g, c2w_ref[...], preferred_element_type=jnp.float32) + c2b_ref[...]


def _tc_mm2_first(h, w1, w2):
    return pl.pallas_call(
        _mm2_first,
        grid=(NB,),
        in_specs=[
            pl.BlockSpec((BN, D), lambda i: (i, 0)),
            pl.BlockSpec((D, H), lambda i: (0, 0)),
            pl.BlockSpec((D, H), lambda i: (0, 0)),
        ],
        out_specs=[
            pl.BlockSpec((BN, H), lambda i: (i, 0)),
            pl.BlockSpec((BN, H), lambda i: (i, 0)),
        ],
        out_shape=[
            jax.ShapeDtypeStruct((N, H), jnp.float32),
            jax.ShapeDtypeStruct((N, H), jnp.float32),
        ],
    )(h, w1, w2)


def _tc_mm2_mid(s_prev, n2, w1, w2):
    return pl.pallas_call(
        _mm2_mid,
        grid=(NB,),
        in_specs=[
            pl.BlockSpec((BN, H), lambda i: (i, 0)),
            pl.BlockSpec((2, BN, HH), lambda i: (0, i, 0)),
            pl.BlockSpec((H, H), lambda i: (0, 0)),
            pl.BlockSpec((H, H), lambda i: (0, 0)),
        ],
        out_specs=[
            pl.BlockSpec((BN, H), lambda i: (i, 0)),
            pl.BlockSpec((BN, H), lambda i: (i, 0)),
        ],
        out_shape=[
            jax.ShapeDtypeStruct((N, H), jnp.float32),
            jax.ShapeDtypeStruct((N, H), jnp.float32),
        ],
    )(s_prev, n2, w1, w2)


def _tc_final(s_prev, n2, batch3, c1w, c1b, c2w, c2b):
    return pl.pallas_call(
        _final,
        grid=(NB,),
        in_specs=[
            pl.BlockSpec((BN, H), lambda i: (i, 0)),
            pl.BlockSpec((2, BN, HH), lambda i: (0, i, 0)),
            pl.BlockSpec((1, 1, BN), lambda i: (i, 0, 0)),
            pl.BlockSpec((H, H), lambda i: (0, 0)),
            pl.BlockSpec((1, H), lambda i: (0, 0)),
            pl.BlockSpec((H, OUT), lambda i: (0, 0)),
            pl.BlockSpec((1, OUT), lambda i: (0, 0)),
        ],
        out_specs=pl.BlockSpec((G, OUT), lambda i: (0, 0)),
        out_shape=jax.ShapeDtypeStruct((G, OUT), jnp.float32),
        scratch_shapes=[pltpu.VMEM((G, H), jnp.float32)],
    )(s_prev, n2, batch3, c1w, c1b, c2w, c2b)


def _sc_agg_body(m2_hbm, pack_hbm, out_hbm,
                 idx_v, rows_v, zbuf, acc_sh,
                 sem_g0, sem_g1, sem_sc0, sem_sc1, sem_idx):
    c = lax.axis_index("c")
    s = lax.axis_index("s")

    zero16 = jnp.zeros((16,), jnp.float32)

    def _zfill(i, _):
        for k in range(HH // 16):
            zbuf[i, pl.ds(k * 16, 16)] = zero16
        return 0

    lax.fori_loop(0, ZROWS, _zfill, 0)

    tbl = m2_hbm
    sem_g = (sem_g0, sem_g1)
    sem_sc = (sem_sc0, sem_sc1)

    pltpu.sync_copy(pack_hbm.at[s, pl.ds(0, KB)], idx_v.at[0])
    pltpu.async_copy(tbl.at[idx_v.at[0, 0, c]], rows_v.at[0], sem_g0)

    def _zdma(j, _):
        pltpu.sync_copy(zbuf, acc_sh.at[pl.ds(s * ZPT + j * ZROWS, ZROWS)])
        return 0

    lax.fori_loop(0, ZPT // ZROWS, _zdma, 0)

    plsc.subcore_barrier()

    def _block(b, _):
        bp = b % 2
        bq = (b + 1) % 2
        for k in range(KB):
            p = k % 2
            q = 1 - p
            if k == 0:
                @pl.when(b > 0)
                def _():
                    pltpu.make_async_copy(
                        rows_v.at[q], acc_sh.at[idx_v.at[bp, k, 2]],
                        sem_sc[q]).wait()
            else:
                pltpu.make_async_copy(
                    rows_v.at[q], acc_sh.at[idx_v.at[bp, k, 2]],
                    sem_sc[q]).wait()
            if k == 1:
                @pl.when(b < NBLK - 1)
                def _():
                    pltpu.async_copy(
                        pack_hbm.at[s, pl.ds((b + 1) * KB, KB)],
                        idx_v.at[bq], sem_idx)
            if k < KB - 1:
                pltpu.async_copy(tbl.at[idx_v.at[bp, k + 1, c]],
                                 rows_v.at[q], sem_g[q])
            else:
                @pl.when(b < NBLK - 1)
                def _():
                    pltpu.make_async_copy(
                        pack_hbm.at[s, pl.ds((b + 1) * KB, KB)],
                        idx_v.at[bq], sem_idx).wait()
                    pltpu.async_copy(tbl.at[idx_v.at[bq, 0, c]],
                                     rows_v.at[q], sem_g[q])
            pltpu.make_async_copy(tbl.at[idx_v.at[bp, k, c]], rows_v.at[p],
                                  sem_g[p]).wait()
            pltpu.async_copy(rows_v.at[p], acc_sh.at[idx_v.at[bp, k, 2]],
                             sem_sc[p], add=True)
        return 0

    lax.fori_loop(0, NBLK, _block, 0)

    lastb = (NBLK - 1) % 2
    pltpu.make_async_copy(rows_v.at[1], acc_sh.at[idx_v.at[lastb, KB - 1, 2]],
                          sem_sc1).wait()

    plsc.subcore_barrier()

    pltpu.sync_copy(acc_sh.at[pl.ds(s * ZPT, ZPT)],
                    out_hbm.at[pl.ds(c * N + s * ZPT, ZPT)])


def _sc_aggregate(m2, pack):
    mesh = plsc.VectorSubcoreMesh(
        core_axis_name="c", subcore_axis_name="s",
        num_cores=NC, num_subcores=NS)
    f = pl.kernel(
        _sc_agg_body,
        out_type=jax.ShapeDtypeStruct((NC * N, HH), jnp.float32),
        mesh=mesh,
        compiler_params=pltpu.CompilerParams(use_tc_tiling_on_sc=False),
        scratch_types=[
            pltpu.VMEM((2, KB, 3, CHUNK), jnp.int32),
            pltpu.VMEM((2, CHUNK, HH), jnp.float32),
            pltpu.VMEM((ZROWS, HH), jnp.float32),
            pltpu.VMEM_SHARED((N, HH), jnp.float32),
            pltpu.SemaphoreType.DMA,
            pltpu.SemaphoreType.DMA,
            pltpu.SemaphoreType.DMA,
            pltpu.SemaphoreType.DMA,
            pltpu.SemaphoreType.DMA,
        ],
    )
    return f(m2, pack)


@jax.jit
def kernel(x, edge_index, batch, W1_0, W2_0, W1_1, W2_1, W1_2, W2_2,
           C1_w, C1_b, C2_w, C2_b):
    src_c = edge_index[0].reshape(NS, NCHUNK, CHUNK)
    pack = jnp.stack([2 * src_c, 2 * src_c + 1,
                      edge_index[1].reshape(NS, NCHUNK, CHUNK)],
                     axis=2)
    batch3 = batch.reshape(NB, 1, BN)
    c1b = C1_b.reshape(1, H)
    c2b = C2_b.reshape(1, OUT)

    s0, m0 = _tc_mm2_first(x, W1_0, W2_0)
    n0 = _sc_aggregate(m0.reshape(2 * N, HH), pack).reshape(2, N, HH)
    s1, m1 = _tc_mm2_mid(s0, n0, W1_1, W2_1)
    n1 = _sc_aggregate(m1.reshape(2 * N, HH), pack).reshape(2, N, HH)
    s2, m2 = _tc_mm2_mid(s1, n1, W1_2, W2_2)
    n2 = _sc_aggregate(m2.reshape(2 * N, HH), pack).reshape(2, N, HH)
    return _tc_final(s2, n2, batch3, C1_w, c1b, C2_w, c2b)
